# baseline (speedup 1.0000x reference)
import functools

import jax
import jax.numpy as jnp
from jax import lax
from jax.experimental import pallas as pl
from jax.experimental.pallas import tpu as pltpu
from jax.experimental.pallas import tpu_sc as plsc

_N = 10000
_E = 320000
_D = 128
_G = 64
_NC, _NS = 2, 16
_NW = _NC * _NS
_CHUNK = 128
_NCH = 80
_EPW = _NCH * _CHUNK
_EPAD = _NW * _EPW
_NPAD = 10240
_DEG_SL = _NPAD // _NS
_RPT = _NPAD // _NS

_mesh = plsc.VectorSubcoreMesh(core_axis_name="c", subcore_axis_name="s")


@functools.partial(
    pl.kernel,
    out_type=jax.ShapeDtypeStruct((_NC, _NPAD), jnp.float32),
    mesh=_mesh,
    scratch_types=[
        pltpu.VMEM((_NCH, _CHUNK), jnp.int32),
        pltpu.VMEM((_CHUNK,), jnp.float32),
        pltpu.VMEM_SHARED((_NPAD,), jnp.float32),
    ],
)
def _deg_kernel(dst_hbm, zeros_hbm, out_hbm, idx_v, ones_v, acc):
    cid = lax.axis_index("c")
    sid = lax.axis_index("s")
    wid = sid * _NC + cid
    sl = pl.ds(sid * _DEG_SL, _DEG_SL)
    pltpu.sync_copy(zeros_hbm.at[sl], acc.at[sl])
    pltpu.sync_copy(dst_hbm.at[wid], idx_v)
    for i in range(_CHUNK // 16):
        ones_v[pl.ds(i * 16, 16)] = jnp.ones((16,), jnp.float32)
    plsc.subcore_barrier()

    def body(j, carry):
        pltpu.sync_copy(ones_v, acc.at[idx_v.at[j]], add=True)
        return carry

    lax.fori_loop(0, _NCH, body, 0)
    plsc.subcore_barrier()
    pltpu.sync_copy(acc.at[sl], out_hbm.at[cid, sl])


@functools.partial(
    pl.kernel,
    out_type=jax.ShapeDtypeStruct((_NC, _NPAD, _D), jnp.float32),
    mesh=_mesh,
    scratch_types=[
        pltpu.VMEM((2, _CHUNK), jnp.int32),
        pltpu.VMEM((2, _CHUNK), jnp.int32),
        pltpu.VMEM((2, _CHUNK), jnp.int32),
        pltpu.VMEM((2, _CHUNK), jnp.int32),
        pltpu.VMEM((_CHUNK, _D), jnp.float32),
        pltpu.VMEM((_CHUNK, _D), jnp.float32),
        pltpu.VMEM_SHARED((_NPAD, _D), jnp.float32),
        pltpu.SemaphoreType.DMA,
        pltpu.SemaphoreType.DMA,
        pltpu.SemaphoreType.DMA,
        pltpu.SemaphoreType.DMA,
        pltpu.SemaphoreType.DMA,
        pltpu.SemaphoreType.DMA,
        pltpu.SemaphoreType.DMA,
        pltpu.SemaphoreType.DMA,
    ],
)
def _agg_kernel(y_hbm, sd_hbm, zeros_hbm, out_hbm,
                i0, i1, i2, i3, rows_a, rows_b, acc,
                ga, gb, sa, sb, p0, p1, p2, p3):
    cid = lax.axis_index("c")
    sid = lax.axis_index("s")
    wid = sid * _NC + cid
    rsl = pl.ds(sid * _RPT, _RPT)
    pltpu.sync_copy(zeros_hbm.at[rsl], acc.at[rsl])

    def gather(ibuf, rbuf, sem):
        pltpu.async_copy(y_hbm.at[ibuf.at[0]], rbuf, sem)

    def gwait(ibuf, rbuf, sem):
        pltpu.make_async_copy(y_hbm.at[ibuf.at[0]], rbuf, sem).wait()

    def scat(ibuf, rbuf, sem):
        pltpu.async_copy(rbuf, acc.at[ibuf.at[1]], sem, add=True)

    def swait(ibuf, rbuf, sem):
        pltpu.make_async_copy(rbuf, acc.at[ibuf.at[1]], sem).wait()

    def prefetch(j, ibuf, sem):
        pltpu.async_copy(sd_hbm.at[wid, j], ibuf, sem)

    def pwait(ibuf, sem):
        pltpu.make_async_copy(sd_hbm.at[wid, 0], ibuf, sem).wait()

    pltpu.sync_copy(sd_hbm.at[wid, 0], i0)
    pltpu.sync_copy(sd_hbm.at[wid, 1], i1)
    prefetch(2, i2, p2)
    prefetch(3, i3, p3)
    gather(i0, rows_a, ga)
    plsc.subcore_barrier()
    gather(i1, rows_b, gb)

    def body(q, carry):
        c = 4 * q
        gwait(i0, rows_a, ga)
        scat(i0, rows_a, sa)
        gwait(i1, rows_b, gb)
        scat(i1, rows_b, sb)
        swait(i0, rows_a, sa)
        prefetch(c + 4, i0, p0)
        pwait(i2, p2)
        gather(i2, rows_a, ga)
        swait(i1, rows_b, sb)
        prefetch(c + 5, i1, p1)
        pwait(i3, p3)
        gather(i3, rows_b, gb)
        gwait(i2, rows_a, ga)
        scat(i2, rows_a, sa)
        gwait(i3, rows_b, gb)
        scat(i3, rows_b, sb)
        swait(i2, rows_a, sa)
        prefetch(c + 6, i2, p2)
        pwait(i0, p0)
        gather(i0, rows_a, ga)
        swait(i3, rows_b, sb)
        prefetch(c + 7, i3, p3)
        pwait(i1, p1)
        gather(i1, rows_b, gb)
        return carry

    lax.fori_loop(0, _NCH // 4 - 1, body, 0)
    gwait(i0, rows_a, ga)
    scat(i0, rows_a, sa)
    gwait(i1, rows_b, gb)
    scat(i1, rows_b, sb)
    swait(i0, rows_a, sa)
    pwait(i2, p2)
    gather(i2, rows_a, ga)
    swait(i1, rows_b, sb)
    pwait(i3, p3)
    gather(i3, rows_b, gb)
    gwait(i2, rows_a, ga)
    scat(i2, rows_a, sa)
    gwait(i3, rows_b, gb)
    scat(i3, rows_b, sb)
    swait(i2, rows_a, sa)
    swait(i3, rows_b, sb)
    plsc.subcore_barrier()
    pltpu.sync_copy(acc.at[rsl], out_hbm.at[cid, rsl])


_BLK = 1000
_NBLK = _N // _BLK


def _tc1_body(x_ref, w_ref, degp_ref, y_ref, dinv_ref):
    deg = degp_ref[0] + degp_ref[1] + 1.0
    dinv = lax.rsqrt(deg)
    xw = jnp.dot(x_ref[...], w_ref[...], preferred_element_type=jnp.float32)
    y_ref[...] = xw * dinv
    dinv_ref[...] = dinv


def _tc1(x, w1, degp_col):
    return pl.pallas_call(
        _tc1_body,
        grid=(_NBLK,),
        in_specs=[
            pl.BlockSpec((_BLK, _D), lambda i: (i, 0)),
            pl.BlockSpec((_D, _D), lambda i: (0, 0)),
            pl.BlockSpec((_NC, _BLK, 1), lambda i: (0, i, 0)),
        ],
        out_specs=[
            pl.BlockSpec((_BLK, _D), lambda i: (i, 0)),
            pl.BlockSpec((_BLK, 1), lambda i: (i, 0)),
        ],
        out_shape=[
            jax.ShapeDtypeStruct((_N, _D), jnp.float32),
            jax.ShapeDtypeStruct((_N, 1), jnp.float32),
        ],
    )(x, w1, degp_col)


def _tc2_body(aggp_ref, y_ref, dinv_ref, b_ref, w_ref, out_ref):
    dinv = dinv_ref[...]
    h = aggp_ref[0] + aggp_ref[1] + y_ref[...]
    h = jnp.maximum(h * dinv + b_ref[...], 0.0)
    out_ref[...] = jnp.dot(h, w_ref[...],
                           preferred_element_type=jnp.float32) * dinv


def _tc2(aggp, y1, dinv, b1, w2):
    return pl.pallas_call(
        _tc2_body,
        grid=(_NBLK,),
        in_specs=[
            pl.BlockSpec((_NC, _BLK, _D), lambda i: (0, i, 0)),
            pl.BlockSpec((_BLK, _D), lambda i: (i, 0)),
            pl.BlockSpec((_BLK, 1), lambda i: (i, 0)),
            pl.BlockSpec((1, _D), lambda i: (0, 0)),
            pl.BlockSpec((_D, _D), lambda i: (0, 0)),
        ],
        out_specs=pl.BlockSpec((_BLK, _D), lambda i: (i, 0)),
        out_shape=jax.ShapeDtypeStruct((_N, _D), jnp.float32),
    )(aggp, y1, dinv, b1, w2)


def _tc3_body(aggp_ref, y_ref, dinv_ref, b_ref, batch_ref, wfc_ref, bfc_ref,
              out_ref, sums_s, cnts_s):
    i = pl.program_id(0)

    @pl.when(i == 0)
    def _():
        sums_s[...] = jnp.zeros_like(sums_s)
        cnts_s[...] = jnp.zeros_like(cnts_s)

    dinv = dinv_ref[...]
    h = aggp_ref[0] + aggp_ref[1] + y_ref[...]
    h = jnp.maximum(h * dinv + b_ref[...], 0.0)
    gids = lax.broadcasted_iota(jnp.int32, (_G, _BLK), 0)
    mask = (gids == batch_ref[0]).astype(jnp.float32)
    sums_s[...] += jnp.dot(mask, h, preferred_element_type=jnp.float32)
    cnts_s[...] += jnp.sum(mask, axis=1, keepdims=True)

    @pl.when(i == _NBLK - 1)
    def _():
        g = sums_s[...] / jnp.maximum(cnts_s[...], 1.0)
        logits = jnp.dot(g, wfc_ref[...],
                         preferred_element_type=jnp.float32) + bfc_ref[...]
        colok = lax.broadcasted_iota(jnp.int32, (_G, _D), 1) < 10
        neg = jnp.where(colok, logits, -1e30)
        m = jnp.max(neg, axis=1, keepdims=True)
        e = jnp.where(colok, jnp.exp(neg - m), 0.0)
        lse = jnp.log(jnp.sum(e, axis=1, keepdims=True)) + m
        out_ref[...] = logits - lse


def _tc3(aggp, y2, dinv, b2, batch3d, wfc_pad, bfc_pad):
    return pl.pallas_call(
        _tc3_body,
        grid=(_NBLK,),
        in_specs=[
            pl.BlockSpec((_NC, _BLK, _D), lambda i: (0, i, 0)),
            pl.BlockSpec((_BLK, _D), lambda i: (i, 0)),
            pl.BlockSpec((_BLK, 1), lambda i: (i, 0)),
            pl.BlockSpec((1, _D), lambda i: (0, 0)),
            pl.BlockSpec((1, 1, _BLK), lambda i: (i, 0, 0)),
            pl.BlockSpec((_D, _D), lambda i: (0, 0)),
            pl.BlockSpec((1, _D), lambda i: (0, 0)),
        ],
        out_specs=pl.BlockSpec((_G, _D), lambda i: (0, 0)),
        out_shape=jax.ShapeDtypeStruct((_G, _D), jnp.float32),
        scratch_shapes=[
            pltpu.VMEM((_G, _D), jnp.float32),
            pltpu.VMEM((_G, 1), jnp.float32),
        ],
    )(aggp, y2, dinv, b2, batch3d, wfc_pad, bfc_pad)


@jax.jit
def _run(x, src, dst, batch, w1, b1, w2, b2, w_fc, b_fc):
    npad_e = _EPAD - _E
    pad_src = (jnp.arange(npad_e, dtype=jnp.int32) * 13) % _N
    pad_dst = _N + (jnp.arange(npad_e, dtype=jnp.int32) % (_NPAD - _N))
    srcp = jnp.concatenate([src, pad_src]).reshape(_NW, _NCH, 1, _CHUNK)
    dstp = jnp.concatenate([dst, pad_dst]).reshape(_NW, _NCH, 1, _CHUNK)
    sd = jnp.concatenate([srcp, dstp], axis=2)
    dst3 = dstp.reshape(_NW, _NCH, _CHUNK)
    zeros1 = jnp.zeros((_NPAD,), jnp.float32)
    zeros2 = jnp.zeros((_NPAD, _D), jnp.float32)

    degp = _deg_kernel(dst3, zeros1)
    degp_col = degp.reshape(_NC, _NPAD, 1)[:, :_N]

    y1, dinv = _tc1(x, w1, degp_col)
    agg1 = _agg_kernel(y1, sd, zeros2)
    y2 = _tc2(agg1, y1, dinv, b1.reshape(1, _D), w2)
    agg2 = _agg_kernel(y2, sd, zeros2)

    wfc_pad = jnp.zeros((_D, _D), jnp.float32).at[:, :10].set(w_fc)
    bfc_pad = jnp.zeros((1, _D), jnp.float32).at[0, :10].set(b_fc)
    batch3d = batch.astype(jnp.int32).reshape(_NBLK, 1, _BLK)
    out = _tc3(agg2, y2, dinv, b2.reshape(1, _D), batch3d, wfc_pad, bfc_pad)
    return out[:, :10]


def kernel(x, edge_index, batch, W1, b1, W2, b2, W_fc, b_fc):
    ei = edge_index.astype(jnp.int32)
    return _run(x, ei[0], ei[1], batch, W1, b1, W2, b2, W_fc, b_fc)


# speedup vs baseline: 1.2011x; 1.2011x over previous
import functools

import jax
import jax.numpy as jnp
from jax import lax
from jax.experimental import pallas as pl
from jax.experimental.pallas import tpu as pltpu
from jax.experimental.pallas import tpu_sc as plsc

_N = 10000
_E = 320000
_D = 128
_G = 64
_NC, _NS = 2, 16
_NW = _NC * _NS
_CHUNK = 128
_NCH = 80
_EPW = _NCH * _CHUNK
_EPAD = _NW * _EPW
_NPAD = 10240
_DEG_SL = _NPAD // _NS
_RPT = _NPAD // _NS

_mesh = plsc.VectorSubcoreMesh(core_axis_name="c", subcore_axis_name="s")


@functools.partial(
    pl.kernel,
    out_type=jax.ShapeDtypeStruct((_NC, _NPAD), jnp.float32),
    mesh=_mesh,
    scratch_types=[
        pltpu.VMEM((_NCH, _CHUNK), jnp.int32),
        pltpu.VMEM((_CHUNK,), jnp.float32),
        pltpu.VMEM_SHARED((_NPAD,), jnp.float32),
    ],
)
def _deg_kernel(dst_hbm, zeros_hbm, out_hbm, idx_v, ones_v, acc):
    cid = lax.axis_index("c")
    sid = lax.axis_index("s")
    wid = sid * _NC + cid
    sl = pl.ds(sid * _DEG_SL, _DEG_SL)
    pltpu.sync_copy(zeros_hbm.at[sl], acc.at[sl])
    pltpu.sync_copy(dst_hbm.at[wid], idx_v)
    for i in range(_CHUNK // 16):
        ones_v[pl.ds(i * 16, 16)] = jnp.ones((16,), jnp.float32)
    plsc.subcore_barrier()

    def body(j, carry):
        pltpu.sync_copy(ones_v, acc.at[idx_v.at[j]], add=True)
        return carry

    lax.fori_loop(0, _NCH, body, 0)
    plsc.subcore_barrier()
    pltpu.sync_copy(acc.at[sl], out_hbm.at[cid, sl])


@functools.partial(
    pl.kernel,
    out_type=jax.ShapeDtypeStruct((_NC, _NPAD, _D), jnp.float32),
    mesh=_mesh,
    scratch_types=[
        pltpu.VMEM((2, _CHUNK), jnp.int32),
        pltpu.VMEM((2, _CHUNK), jnp.int32),
        pltpu.VMEM((2, _CHUNK), jnp.int32),
        pltpu.VMEM((2, _CHUNK), jnp.int32),
        pltpu.VMEM((_CHUNK, _D), jnp.float32),
        pltpu.VMEM((_CHUNK, _D), jnp.float32),
        pltpu.VMEM_SHARED((_NPAD, _D), jnp.float32),
        pltpu.SemaphoreType.DMA,
        pltpu.SemaphoreType.DMA,
        pltpu.SemaphoreType.DMA,
        pltpu.SemaphoreType.DMA,
        pltpu.SemaphoreType.DMA,
        pltpu.SemaphoreType.DMA,
    ],
)
def _agg_kernel(y_hbm, sd_hbm, zeros_hbm, out_hbm,
                i0, i1, i2, i3, rows_a, rows_b, acc,
                ga, gb, p0, p1, p2, p3):
    cid = lax.axis_index("c")
    sid = lax.axis_index("s")
    wid = sid * _NC + cid
    rsl = pl.ds(sid * _RPT, _RPT)
    pltpu.sync_copy(zeros_hbm.at[rsl], acc.at[rsl])

    def gather(ibuf, rbuf, sem):
        pltpu.async_copy(y_hbm.at[ibuf.at[0]], rbuf, sem)

    def gwait(ibuf, rbuf, sem):
        pltpu.make_async_copy(y_hbm.at[ibuf.at[0]], rbuf, sem).wait()

    def scat(ibuf, rbuf):
        pltpu.sync_copy(rbuf, acc.at[ibuf.at[1]], add=True)

    def prefetch(j, ibuf, sem):
        pltpu.async_copy(sd_hbm.at[wid, j], ibuf, sem)

    def pwait(ibuf, sem):
        pltpu.make_async_copy(sd_hbm.at[wid, 0], ibuf, sem).wait()

    pltpu.sync_copy(sd_hbm.at[wid, 0], i0)
    pltpu.sync_copy(sd_hbm.at[wid, 1], i1)
    prefetch(2, i2, p2)
    prefetch(3, i3, p3)
    gather(i0, rows_a, ga)
    plsc.subcore_barrier()
    gather(i1, rows_b, gb)

    def body(q, carry):
        c = 4 * q
        gwait(i0, rows_a, ga)
        scat(i0, rows_a)
        prefetch(c + 4, i0, p0)
        gwait(i1, rows_b, gb)
        pwait(i2, p2)
        gather(i2, rows_a, ga)
        scat(i1, rows_b)
        prefetch(c + 5, i1, p1)
        pwait(i3, p3)
        gather(i3, rows_b, gb)
        gwait(i2, rows_a, ga)
        scat(i2, rows_a)
        prefetch(c + 6, i2, p2)
        gwait(i3, rows_b, gb)
        pwait(i0, p0)
        gather(i0, rows_a, ga)
        scat(i3, rows_b)
        prefetch(c + 7, i3, p3)
        pwait(i1, p1)
        gather(i1, rows_b, gb)
        return carry

    lax.fori_loop(0, _NCH // 4 - 1, body, 0)
    gwait(i0, rows_a, ga)
    scat(i0, rows_a)
    gwait(i1, rows_b, gb)
    pwait(i2, p2)
    gather(i2, rows_a, ga)
    scat(i1, rows_b)
    pwait(i3, p3)
    gather(i3, rows_b, gb)
    gwait(i2, rows_a, ga)
    scat(i2, rows_a)
    gwait(i3, rows_b, gb)
    scat(i3, rows_b)
    plsc.subcore_barrier()
    pltpu.sync_copy(acc.at[rsl], out_hbm.at[cid, rsl])


_BLK = 1000
_NBLK = _N // _BLK


def _tc1_body(x_ref, w_ref, degp_ref, y_ref, dinv_ref):
    deg = degp_ref[0] + degp_ref[1] + 1.0
    dinv = lax.rsqrt(deg)
    xw = jnp.dot(x_ref[...], w_ref[...], preferred_element_type=jnp.float32)
    y_ref[...] = xw * dinv
    dinv_ref[...] = dinv


def _tc1(x, w1, degp_col):
    return pl.pallas_call(
        _tc1_body,
        grid=(_NBLK,),
        in_specs=[
            pl.BlockSpec((_BLK, _D), lambda i: (i, 0)),
            pl.BlockSpec((_D, _D), lambda i: (0, 0)),
            pl.BlockSpec((_NC, _BLK, 1), lambda i: (0, i, 0)),
        ],
        out_specs=[
            pl.BlockSpec((_BLK, _D), lambda i: (i, 0)),
            pl.BlockSpec((_BLK, 1), lambda i: (i, 0)),
        ],
        out_shape=[
            jax.ShapeDtypeStruct((_N, _D), jnp.float32),
            jax.ShapeDtypeStruct((_N, 1), jnp.float32),
        ],
    )(x, w1, degp_col)


def _tc2_body(aggp_ref, y_ref, dinv_ref, b_ref, w_ref, out_ref):
    dinv = dinv_ref[...]
    h = aggp_ref[0] + aggp_ref[1] + y_ref[...]
    h = jnp.maximum(h * dinv + b_ref[...], 0.0)
    out_ref[...] = jnp.dot(h, w_ref[...],
                           preferred_element_type=jnp.float32) * dinv


def _tc2(aggp, y1, dinv, b1, w2):
    return pl.pallas_call(
        _tc2_body,
        grid=(_NBLK,),
        in_specs=[
            pl.BlockSpec((_NC, _BLK, _D), lambda i: (0, i, 0)),
            pl.BlockSpec((_BLK, _D), lambda i: (i, 0)),
            pl.BlockSpec((_BLK, 1), lambda i: (i, 0)),
            pl.BlockSpec((1, _D), lambda i: (0, 0)),
            pl.BlockSpec((_D, _D), lambda i: (0, 0)),
        ],
        out_specs=pl.BlockSpec((_BLK, _D), lambda i: (i, 0)),
        out_shape=jax.ShapeDtypeStruct((_N, _D), jnp.float32),
    )(aggp, y1, dinv, b1, w2)


def _tc3_body(aggp_ref, y_ref, dinv_ref, b_ref, batch_ref, wfc_ref, bfc_ref,
              out_ref, sums_s, cnts_s):
    i = pl.program_id(0)

    @pl.when(i == 0)
    def _():
        sums_s[...] = jnp.zeros_like(sums_s)
        cnts_s[...] = jnp.zeros_like(cnts_s)

    dinv = dinv_ref[...]
    h = aggp_ref[0] + aggp_ref[1] + y_ref[...]
    h = jnp.maximum(h * dinv + b_ref[...], 0.0)
    gids = lax.broadcasted_iota(jnp.int32, (_G, _BLK), 0)
    mask = (gids == batch_ref[0]).astype(jnp.float32)
    sums_s[...] += jnp.dot(mask, h, preferred_element_type=jnp.float32)
    cnts_s[...] += jnp.sum(mask, axis=1, keepdims=True)

    @pl.when(i == _NBLK - 1)
    def _():
        g = sums_s[...] / jnp.maximum(cnts_s[...], 1.0)
        logits = jnp.dot(g, wfc_ref[...],
                         preferred_element_type=jnp.float32) + bfc_ref[...]
        colok = lax.broadcasted_iota(jnp.int32, (_G, _D), 1) < 10
        neg = jnp.where(colok, logits, -1e30)
        m = jnp.max(neg, axis=1, keepdims=True)
        e = jnp.where(colok, jnp.exp(neg - m), 0.0)
        lse = jnp.log(jnp.sum(e, axis=1, keepdims=True)) + m
        out_ref[...] = logits - lse


def _tc3(aggp, y2, dinv, b2, batch3d, wfc_pad, bfc_pad):
    return pl.pallas_call(
        _tc3_body,
        grid=(_NBLK,),
        in_specs=[
            pl.BlockSpec((_NC, _BLK, _D), lambda i: (0, i, 0)),
            pl.BlockSpec((_BLK, _D), lambda i: (i, 0)),
            pl.BlockSpec((_BLK, 1), lambda i: (i, 0)),
            pl.BlockSpec((1, _D), lambda i: (0, 0)),
            pl.BlockSpec((1, 1, _BLK), lambda i: (i, 0, 0)),
            pl.BlockSpec((_D, _D), lambda i: (0, 0)),
            pl.BlockSpec((1, _D), lambda i: (0, 0)),
        ],
        out_specs=pl.BlockSpec((_G, _D), lambda i: (0, 0)),
        out_shape=jax.ShapeDtypeStruct((_G, _D), jnp.float32),
        scratch_shapes=[
            pltpu.VMEM((_G, _D), jnp.float32),
            pltpu.VMEM((_G, 1), jnp.float32),
        ],
    )(aggp, y2, dinv, b2, batch3d, wfc_pad, bfc_pad)


@jax.jit
def _run(x, src, dst, batch, w1, b1, w2, b2, w_fc, b_fc):
    npad_e = _EPAD - _E
    pad_src = (jnp.arange(npad_e, dtype=jnp.int32) * 13) % _N
    pad_dst = _N + (jnp.arange(npad_e, dtype=jnp.int32) % (_NPAD - _N))
    srcp = jnp.concatenate([src, pad_src]).reshape(_NW, _NCH, 1, _CHUNK)
    dstp = jnp.concatenate([dst, pad_dst]).reshape(_NW, _NCH, 1, _CHUNK)
    sd = jnp.concatenate([srcp, dstp], axis=2)
    dst3 = dstp.reshape(_NW, _NCH, _CHUNK)
    zeros1 = jnp.zeros((_NPAD,), jnp.float32)
    zeros2 = jnp.zeros((_NPAD, _D), jnp.float32)

    degp = _deg_kernel(dst3, zeros1)
    degp_col = degp.reshape(_NC, _NPAD, 1)[:, :_N]

    y1, dinv = _tc1(x, w1, degp_col)
    agg1 = _agg_kernel(y1, sd, zeros2)
    y2 = _tc2(agg1, y1, dinv, b1.reshape(1, _D), w2)
    agg2 = _agg_kernel(y2, sd, zeros2)

    wfc_pad = jnp.zeros((_D, _D), jnp.float32).at[:, :10].set(w_fc)
    bfc_pad = jnp.zeros((1, _D), jnp.float32).at[0, :10].set(b_fc)
    batch3d = batch.astype(jnp.int32).reshape(_NBLK, 1, _BLK)
    out = _tc3(agg2, y2, dinv, b2.reshape(1, _D), batch3d, wfc_pad, bfc_pad)
    return out[:, :10]


def kernel(x, edge_index, batch, W1, b1, W2, b2, W_fc, b_fc):
    ei = edge_index.astype(jnp.int32)
    return _run(x, ei[0], ei[1], batch, W1, b1, W2, b2, W_fc, b_fc)


# speedup vs baseline: 1.2042x; 1.0026x over previous
import functools

import jax
import jax.numpy as jnp
from jax import lax
from jax.experimental import pallas as pl
from jax.experimental.pallas import tpu as pltpu
from jax.experimental.pallas import tpu_sc as plsc

_N = 10000
_E = 320000
_D = 128
_G = 64
_NC, _NS = 2, 16
_NW = _NC * _NS
_CHUNK = 128
_NCH = 80
_EPW = _NCH * _CHUNK
_EPAD = _NW * _EPW
_NPAD = 10240
_DEG_SL = _NPAD // _NS
_RPT = _NPAD // _NS

_mesh = plsc.VectorSubcoreMesh(core_axis_name="c", subcore_axis_name="s")


@functools.partial(
    pl.kernel,
    out_type=jax.ShapeDtypeStruct((_NC, _NPAD), jnp.float32),
    mesh=_mesh,
    scratch_types=[
        pltpu.VMEM((_NCH, _CHUNK), jnp.int32),
        pltpu.VMEM((_CHUNK,), jnp.float32),
        pltpu.VMEM_SHARED((_NPAD,), jnp.float32),
    ],
)
def _deg_kernel(dst_hbm, zeros_hbm, out_hbm, idx_v, ones_v, acc):
    cid = lax.axis_index("c")
    sid = lax.axis_index("s")
    wid = sid * _NC + cid
    sl = pl.ds(sid * _DEG_SL, _DEG_SL)
    pltpu.sync_copy(zeros_hbm.at[sl], acc.at[sl])
    pltpu.sync_copy(dst_hbm.at[wid], idx_v)
    for i in range(_CHUNK // 16):
        ones_v[pl.ds(i * 16, 16)] = jnp.ones((16,), jnp.float32)
    plsc.subcore_barrier()

    def body(j, carry):
        pltpu.sync_copy(ones_v, acc.at[idx_v.at[j]], add=True)
        return carry

    lax.fori_loop(0, _NCH, body, 0)
    plsc.subcore_barrier()
    pltpu.sync_copy(acc.at[sl], out_hbm.at[cid, sl])


@functools.partial(
    pl.kernel,
    out_type=jax.ShapeDtypeStruct((_NC, _NPAD, _D), jnp.float32),
    mesh=_mesh,
    scratch_types=[
        pltpu.VMEM((2, _CHUNK), jnp.int32),
        pltpu.VMEM((2, _CHUNK), jnp.int32),
        pltpu.VMEM((2, _CHUNK), jnp.int32),
        pltpu.VMEM((2, _CHUNK), jnp.int32),
        pltpu.VMEM((_CHUNK, _D), jnp.float32),
        pltpu.VMEM((_CHUNK, _D), jnp.float32),
        pltpu.VMEM_SHARED((_NPAD, _D), jnp.float32),
        pltpu.SemaphoreType.DMA,
        pltpu.SemaphoreType.DMA,
        pltpu.SemaphoreType.DMA,
        pltpu.SemaphoreType.DMA,
        pltpu.SemaphoreType.DMA,
        pltpu.SemaphoreType.DMA,
    ],
)
def _agg_kernel(y_hbm, sd_hbm, zeros_hbm, out_hbm,
                i0, i1, i2, i3, rows_a, rows_b, acc,
                ga, gb, p0, p1, p2, p3):
    cid = lax.axis_index("c")
    sid = lax.axis_index("s")
    wid = sid * _NC + cid
    rsl = pl.ds(sid * _RPT, _RPT)
    pltpu.sync_copy(zeros_hbm.at[rsl], acc.at[rsl])

    def gather(ibuf, rbuf, sem):
        pltpu.async_copy(y_hbm.at[ibuf.at[0]], rbuf, sem)

    def gwait(ibuf, rbuf, sem):
        pltpu.make_async_copy(y_hbm.at[ibuf.at[0]], rbuf, sem).wait()

    def scat(ibuf, rbuf):
        pltpu.sync_copy(rbuf, acc.at[ibuf.at[1]], add=True)

    def prefetch(j, ibuf, sem):
        pltpu.async_copy(sd_hbm.at[wid, j], ibuf, sem)

    def pwait(ibuf, sem):
        pltpu.make_async_copy(sd_hbm.at[wid, 0], ibuf, sem).wait()

    pltpu.sync_copy(sd_hbm.at[wid, 0], i0)
    pltpu.sync_copy(sd_hbm.at[wid, 1], i1)
    prefetch(2, i2, p2)
    prefetch(3, i3, p3)
    gather(i0, rows_a, ga)
    plsc.subcore_barrier()
    gather(i1, rows_b, gb)

    def body(q, carry):
        c = 4 * q
        gwait(i0, rows_a, ga)
        scat(i0, rows_a)
        prefetch(c + 4, i0, p0)
        gwait(i1, rows_b, gb)
        pwait(i2, p2)
        gather(i2, rows_a, ga)
        scat(i1, rows_b)
        prefetch(c + 5, i1, p1)
        pwait(i3, p3)
        gather(i3, rows_b, gb)
        gwait(i2, rows_a, ga)
        scat(i2, rows_a)
        prefetch(c + 6, i2, p2)
        gwait(i3, rows_b, gb)
        pwait(i0, p0)
        gather(i0, rows_a, ga)
        scat(i3, rows_b)
        prefetch(c + 7, i3, p3)
        pwait(i1, p1)
        gather(i1, rows_b, gb)
        return carry

    lax.fori_loop(0, _NCH // 4 - 1, body, 0)
    gwait(i0, rows_a, ga)
    scat(i0, rows_a)
    gwait(i1, rows_b, gb)
    pwait(i2, p2)
    gather(i2, rows_a, ga)
    scat(i1, rows_b)
    pwait(i3, p3)
    gather(i3, rows_b, gb)
    gwait(i2, rows_a, ga)
    scat(i2, rows_a)
    gwait(i3, rows_b, gb)
    scat(i3, rows_b)
    plsc.subcore_barrier()
    pltpu.sync_copy(acc.at[rsl], out_hbm.at[cid, rsl])


_BLK = 1000
_NBLK = _N // _BLK


def _tc1a_body(x_ref, w_ref, xw_ref):
    xw_ref[...] = jnp.dot(x_ref[...], w_ref[...],
                          preferred_element_type=jnp.float32)


def _tc1a(x, w1):
    return pl.pallas_call(
        _tc1a_body,
        grid=(_NBLK,),
        in_specs=[
            pl.BlockSpec((_BLK, _D), lambda i: (i, 0)),
            pl.BlockSpec((_D, _D), lambda i: (0, 0)),
        ],
        out_specs=pl.BlockSpec((_BLK, _D), lambda i: (i, 0)),
        out_shape=jax.ShapeDtypeStruct((_N, _D), jnp.float32),
    )(x, w1)


def _tc1b_body(xw_ref, degp_ref, y_ref, dinv_ref):
    deg = degp_ref[0] + degp_ref[1] + 1.0
    dinv = lax.rsqrt(deg)
    y_ref[...] = xw_ref[...] * dinv
    dinv_ref[...] = dinv


def _tc1b(xw, degp_col):
    return pl.pallas_call(
        _tc1b_body,
        grid=(_NBLK,),
        in_specs=[
            pl.BlockSpec((_BLK, _D), lambda i: (i, 0)),
            pl.BlockSpec((_NC, _BLK, 1), lambda i: (0, i, 0)),
        ],
        out_specs=[
            pl.BlockSpec((_BLK, _D), lambda i: (i, 0)),
            pl.BlockSpec((_BLK, 1), lambda i: (i, 0)),
        ],
        out_shape=[
            jax.ShapeDtypeStruct((_N, _D), jnp.float32),
            jax.ShapeDtypeStruct((_N, 1), jnp.float32),
        ],
    )(xw, degp_col)


def _tc2_body(aggp_ref, y_ref, dinv_ref, b_ref, w_ref, out_ref):
    dinv = dinv_ref[...]
    h = aggp_ref[0] + aggp_ref[1] + y_ref[...]
    h = jnp.maximum(h * dinv + b_ref[...], 0.0)
    out_ref[...] = jnp.dot(h, w_ref[...],
                           preferred_element_type=jnp.float32) * dinv


def _tc2(aggp, y1, dinv, b1, w2):
    return pl.pallas_call(
        _tc2_body,
        grid=(_NBLK,),
        in_specs=[
            pl.BlockSpec((_NC, _BLK, _D), lambda i: (0, i, 0)),
            pl.BlockSpec((_BLK, _D), lambda i: (i, 0)),
            pl.BlockSpec((_BLK, 1), lambda i: (i, 0)),
            pl.BlockSpec((1, _D), lambda i: (0, 0)),
            pl.BlockSpec((_D, _D), lambda i: (0, 0)),
        ],
        out_specs=pl.BlockSpec((_BLK, _D), lambda i: (i, 0)),
        out_shape=jax.ShapeDtypeStruct((_N, _D), jnp.float32),
    )(aggp, y1, dinv, b1, w2)


def _tc3_body(aggp_ref, y_ref, dinv_ref, b_ref, batch_ref, wfc_ref, bfc_ref,
              out_ref, sums_s, cnts_s):
    i = pl.program_id(0)

    @pl.when(i == 0)
    def _():
        sums_s[...] = jnp.zeros_like(sums_s)
        cnts_s[...] = jnp.zeros_like(cnts_s)

    dinv = dinv_ref[...]
    h = aggp_ref[0] + aggp_ref[1] + y_ref[...]
    h = jnp.maximum(h * dinv + b_ref[...], 0.0)
    gids = lax.broadcasted_iota(jnp.int32, (_G, _BLK), 0)
    mask = (gids == batch_ref[0]).astype(jnp.float32)
    sums_s[...] += jnp.dot(mask, h, preferred_element_type=jnp.float32)
    cnts_s[...] += jnp.sum(mask, axis=1, keepdims=True)

    @pl.when(i == _NBLK - 1)
    def _():
        g = sums_s[...] / jnp.maximum(cnts_s[...], 1.0)
        logits = jnp.dot(g, wfc_ref[...],
                         preferred_element_type=jnp.float32) + bfc_ref[...]
        colok = lax.broadcasted_iota(jnp.int32, (_G, _D), 1) < 10
        neg = jnp.where(colok, logits, -1e30)
        m = jnp.max(neg, axis=1, keepdims=True)
        e = jnp.where(colok, jnp.exp(neg - m), 0.0)
        lse = jnp.log(jnp.sum(e, axis=1, keepdims=True)) + m
        out_ref[...] = logits - lse


def _tc3(aggp, y2, dinv, b2, batch3d, wfc_pad, bfc_pad):
    return pl.pallas_call(
        _tc3_body,
        grid=(_NBLK,),
        in_specs=[
            pl.BlockSpec((_NC, _BLK, _D), lambda i: (0, i, 0)),
            pl.BlockSpec((_BLK, _D), lambda i: (i, 0)),
            pl.BlockSpec((_BLK, 1), lambda i: (i, 0)),
            pl.BlockSpec((1, _D), lambda i: (0, 0)),
            pl.BlockSpec((1, 1, _BLK), lambda i: (i, 0, 0)),
            pl.BlockSpec((_D, _D), lambda i: (0, 0)),
            pl.BlockSpec((1, _D), lambda i: (0, 0)),
        ],
        out_specs=pl.BlockSpec((_G, _D), lambda i: (0, 0)),
        out_shape=jax.ShapeDtypeStruct((_G, _D), jnp.float32),
        scratch_shapes=[
            pltpu.VMEM((_G, _D), jnp.float32),
            pltpu.VMEM((_G, 1), jnp.float32),
        ],
    )(aggp, y2, dinv, b2, batch3d, wfc_pad, bfc_pad)


@jax.jit
def _run(x, src, dst, batch, w1, b1, w2, b2, w_fc, b_fc):
    npad_e = _EPAD - _E
    pad_src = (jnp.arange(npad_e, dtype=jnp.int32) * 13) % _N
    pad_dst = _N + (jnp.arange(npad_e, dtype=jnp.int32) % (_NPAD - _N))
    srcp = jnp.concatenate([src, pad_src]).reshape(_NW, _NCH, 1, _CHUNK)
    dstp = jnp.concatenate([dst, pad_dst]).reshape(_NW, _NCH, 1, _CHUNK)
    sd = jnp.concatenate([srcp, dstp], axis=2)
    dst3 = dstp.reshape(_NW, _NCH, _CHUNK)
    zeros1 = jnp.zeros((_NPAD,), jnp.float32)
    zeros2 = jnp.zeros((_NPAD, _D), jnp.float32)

    degp = _deg_kernel(dst3, zeros1)
    degp_col = degp.reshape(_NC, _NPAD, 1)[:, :_N]

    xw1 = _tc1a(x, w1)
    y1, dinv = _tc1b(xw1, degp_col)
    agg1 = _agg_kernel(y1, sd, zeros2)
    y2 = _tc2(agg1, y1, dinv, b1.reshape(1, _D), w2)
    agg2 = _agg_kernel(y2, sd, zeros2)

    wfc_pad = jnp.zeros((_D, _D), jnp.float32).at[:, :10].set(w_fc)
    bfc_pad = jnp.zeros((1, _D), jnp.float32).at[0, :10].set(b_fc)
    batch3d = batch.astype(jnp.int32).reshape(_NBLK, 1, _BLK)
    out = _tc3(agg2, y2, dinv, b2.reshape(1, _D), batch3d, wfc_pad, bfc_pad)
    return out[:, :10]


def kernel(x, edge_index, batch, W1, b1, W2, b2, W_fc, b_fc):
    ei = edge_index.astype(jnp.int32)
    return _run(x, ei[0], ei[1], batch, W1, b1, W2, b2, W_fc, b_fc)


# speedup vs baseline: 1.2042x; 1.0000x over previous
import functools

import jax
import jax.numpy as jnp
from jax import lax
from jax.experimental import pallas as pl
from jax.experimental.pallas import tpu as pltpu
from jax.experimental.pallas import tpu_sc as plsc

_N = 10000
_E = 320000
_D = 128
_G = 64
_NC, _NS = 2, 16
_NW = _NC * _NS
_CHUNK = 128
_NCH = 80
_EPW = _NCH * _CHUNK
_EPAD = _NW * _EPW
_NPAD = 10240
_DEG_SL = _NPAD // _NS
_RPT = _NPAD // _NS

_mesh = plsc.VectorSubcoreMesh(core_axis_name="c", subcore_axis_name="s")


@functools.partial(
    pl.kernel,
    out_type=jax.ShapeDtypeStruct((_NC, _NPAD), jnp.float32),
    mesh=_mesh,
    scratch_types=[
        pltpu.VMEM((_NCH, _CHUNK), jnp.int32),
        pltpu.VMEM((_CHUNK,), jnp.float32),
        pltpu.VMEM_SHARED((_NPAD,), jnp.float32),
        pltpu.SemaphoreType.DMA,
        pltpu.SemaphoreType.DMA,
        pltpu.SemaphoreType.DMA,
        pltpu.SemaphoreType.DMA,
    ],
)
def _deg_kernel(dst_hbm, zeros_hbm, out_hbm, idx_v, ones_v, acc, s0, s1, s2, s3):
    cid = lax.axis_index("c")
    sid = lax.axis_index("s")
    wid = sid * _NC + cid
    sl = pl.ds(sid * _DEG_SL, _DEG_SL)
    pltpu.sync_copy(zeros_hbm.at[sl], acc.at[sl])
    pltpu.sync_copy(dst_hbm.at[wid], idx_v)
    for i in range(_CHUNK // 16):
        ones_v[pl.ds(i * 16, 16)] = jnp.ones((16,), jnp.float32)
    plsc.subcore_barrier()

    def scat(j, sem):
        pltpu.async_copy(ones_v, acc.at[idx_v.at[j]], sem, add=True)

    def swait(sem):
        pltpu.make_async_copy(ones_v, acc.at[idx_v.at[0]], sem).wait()

    sems = (s0, s1, s2, s3)
    for k in range(4):
        scat(k, sems[k])

    def body(i, carry):
        j = 4 * i
        for k in range(4):
            swait(sems[k])
            scat(j + 4 + k, sems[k])
        return carry

    lax.fori_loop(0, _NCH // 4 - 1, body, 0)
    for k in range(4):
        swait(sems[k])
    plsc.subcore_barrier()
    pltpu.sync_copy(acc.at[sl], out_hbm.at[cid, sl])


@functools.partial(
    pl.kernel,
    out_type=jax.ShapeDtypeStruct((_NC, _NPAD, _D), jnp.float32),
    mesh=_mesh,
    scratch_types=[
        pltpu.VMEM((2, _CHUNK), jnp.int32),
        pltpu.VMEM((2, _CHUNK), jnp.int32),
        pltpu.VMEM((2, _CHUNK), jnp.int32),
        pltpu.VMEM((2, _CHUNK), jnp.int32),
        pltpu.VMEM((_CHUNK, _D), jnp.float32),
        pltpu.VMEM((_CHUNK, _D), jnp.float32),
        pltpu.VMEM_SHARED((_NPAD, _D), jnp.float32),
        pltpu.SemaphoreType.DMA,
        pltpu.SemaphoreType.DMA,
        pltpu.SemaphoreType.DMA,
        pltpu.SemaphoreType.DMA,
        pltpu.SemaphoreType.DMA,
        pltpu.SemaphoreType.DMA,
    ],
)
def _agg_kernel(y_hbm, sd_hbm, zeros_hbm, out_hbm,
                i0, i1, i2, i3, rows_a, rows_b, acc,
                ga, gb, p0, p1, p2, p3):
    cid = lax.axis_index("c")
    sid = lax.axis_index("s")
    wid = sid * _NC + cid
    rsl = pl.ds(sid * _RPT, _RPT)
    pltpu.sync_copy(zeros_hbm.at[rsl], acc.at[rsl])

    def gather(ibuf, rbuf, sem):
        pltpu.async_copy(y_hbm.at[ibuf.at[0]], rbuf, sem)

    def gwait(ibuf, rbuf, sem):
        pltpu.make_async_copy(y_hbm.at[ibuf.at[0]], rbuf, sem).wait()

    def scat(ibuf, rbuf):
        pltpu.sync_copy(rbuf, acc.at[ibuf.at[1]], add=True)

    def prefetch(j, ibuf, sem):
        pltpu.async_copy(sd_hbm.at[wid, j], ibuf, sem)

    def pwait(ibuf, sem):
        pltpu.make_async_copy(sd_hbm.at[wid, 0], ibuf, sem).wait()

    pltpu.sync_copy(sd_hbm.at[wid, 0], i0)
    pltpu.sync_copy(sd_hbm.at[wid, 1], i1)
    prefetch(2, i2, p2)
    prefetch(3, i3, p3)
    gather(i0, rows_a, ga)
    plsc.subcore_barrier()
    gather(i1, rows_b, gb)

    def body(q, carry):
        c = 4 * q
        gwait(i0, rows_a, ga)
        scat(i0, rows_a)
        prefetch(c + 4, i0, p0)
        gwait(i1, rows_b, gb)
        pwait(i2, p2)
        gather(i2, rows_a, ga)
        scat(i1, rows_b)
        prefetch(c + 5, i1, p1)
        pwait(i3, p3)
        gather(i3, rows_b, gb)
        gwait(i2, rows_a, ga)
        scat(i2, rows_a)
        prefetch(c + 6, i2, p2)
        gwait(i3, rows_b, gb)
        pwait(i0, p0)
        gather(i0, rows_a, ga)
        scat(i3, rows_b)
        prefetch(c + 7, i3, p3)
        pwait(i1, p1)
        gather(i1, rows_b, gb)
        return carry

    lax.fori_loop(0, _NCH // 4 - 1, body, 0)
    gwait(i0, rows_a, ga)
    scat(i0, rows_a)
    gwait(i1, rows_b, gb)
    pwait(i2, p2)
    gather(i2, rows_a, ga)
    scat(i1, rows_b)
    pwait(i3, p3)
    gather(i3, rows_b, gb)
    gwait(i2, rows_a, ga)
    scat(i2, rows_a)
    gwait(i3, rows_b, gb)
    scat(i3, rows_b)
    plsc.subcore_barrier()
    pltpu.sync_copy(acc.at[rsl], out_hbm.at[cid, rsl])


_BLK = 1000
_NBLK = _N // _BLK


def _tc1a_body(x_ref, w_ref, xw_ref):
    xw_ref[...] = jnp.dot(x_ref[...], w_ref[...],
                          preferred_element_type=jnp.float32)


def _tc1a(x, w1):
    return pl.pallas_call(
        _tc1a_body,
        grid=(_NBLK,),
        in_specs=[
            pl.BlockSpec((_BLK, _D), lambda i: (i, 0)),
            pl.BlockSpec((_D, _D), lambda i: (0, 0)),
        ],
        out_specs=pl.BlockSpec((_BLK, _D), lambda i: (i, 0)),
        out_shape=jax.ShapeDtypeStruct((_N, _D), jnp.float32),
    )(x, w1)


def _tc1b_body(xw_ref, degp_ref, y_ref, dinv_ref):
    deg = degp_ref[0] + degp_ref[1] + 1.0
    dinv = lax.rsqrt(deg)
    y_ref[...] = xw_ref[...] * dinv
    dinv_ref[...] = dinv


def _tc1b(xw, degp_col):
    return pl.pallas_call(
        _tc1b_body,
        grid=(_NBLK,),
        in_specs=[
            pl.BlockSpec((_BLK, _D), lambda i: (i, 0)),
            pl.BlockSpec((_NC, _BLK, 1), lambda i: (0, i, 0)),
        ],
        out_specs=[
            pl.BlockSpec((_BLK, _D), lambda i: (i, 0)),
            pl.BlockSpec((_BLK, 1), lambda i: (i, 0)),
        ],
        out_shape=[
            jax.ShapeDtypeStruct((_N, _D), jnp.float32),
            jax.ShapeDtypeStruct((_N, 1), jnp.float32),
        ],
    )(xw, degp_col)


def _tc2_body(aggp_ref, y_ref, dinv_ref, b_ref, w_ref, out_ref):
    dinv = dinv_ref[...]
    h = aggp_ref[0] + aggp_ref[1] + y_ref[...]
    h = jnp.maximum(h * dinv + b_ref[...], 0.0)
    out_ref[...] = jnp.dot(h, w_ref[...],
                           preferred_element_type=jnp.float32) * dinv


def _tc2(aggp, y1, dinv, b1, w2):
    return pl.pallas_call(
        _tc2_body,
        grid=(_NBLK,),
        in_specs=[
            pl.BlockSpec((_NC, _BLK, _D), lambda i: (0, i, 0)),
            pl.BlockSpec((_BLK, _D), lambda i: (i, 0)),
            pl.BlockSpec((_BLK, 1), lambda i: (i, 0)),
            pl.BlockSpec((1, _D), lambda i: (0, 0)),
            pl.BlockSpec((_D, _D), lambda i: (0, 0)),
        ],
        out_specs=pl.BlockSpec((_BLK, _D), lambda i: (i, 0)),
        out_shape=jax.ShapeDtypeStruct((_N, _D), jnp.float32),
    )(aggp, y1, dinv, b1, w2)


def _tc3_body(aggp_ref, y_ref, dinv_ref, b_ref, batch_ref, wfc_ref, bfc_ref,
              out_ref, sums_s, cnts_s):
    i = pl.program_id(0)

    @pl.when(i == 0)
    def _():
        sums_s[...] = jnp.zeros_like(sums_s)
        cnts_s[...] = jnp.zeros_like(cnts_s)

    dinv = dinv_ref[...]
    h = aggp_ref[0] + aggp_ref[1] + y_ref[...]
    h = jnp.maximum(h * dinv + b_ref[...], 0.0)
    gids = lax.broadcasted_iota(jnp.int32, (_G, _BLK), 0)
    mask = (gids == batch_ref[0]).astype(jnp.float32)
    sums_s[...] += jnp.dot(mask, h, preferred_element_type=jnp.float32)
    cnts_s[...] += jnp.sum(mask, axis=1, keepdims=True)

    @pl.when(i == _NBLK - 1)
    def _():
        g = sums_s[...] / jnp.maximum(cnts_s[...], 1.0)
        logits = jnp.dot(g, wfc_ref[...],
                         preferred_element_type=jnp.float32) + bfc_ref[...]
        colok = lax.broadcasted_iota(jnp.int32, (_G, _D), 1) < 10
        neg = jnp.where(colok, logits, -1e30)
        m = jnp.max(neg, axis=1, keepdims=True)
        e = jnp.where(colok, jnp.exp(neg - m), 0.0)
        lse = jnp.log(jnp.sum(e, axis=1, keepdims=True)) + m
        out_ref[...] = logits - lse


def _tc3(aggp, y2, dinv, b2, batch3d, wfc_pad, bfc_pad):
    return pl.pallas_call(
        _tc3_body,
        grid=(_NBLK,),
        in_specs=[
            pl.BlockSpec((_NC, _BLK, _D), lambda i: (0, i, 0)),
            pl.BlockSpec((_BLK, _D), lambda i: (i, 0)),
            pl.BlockSpec((_BLK, 1), lambda i: (i, 0)),
            pl.BlockSpec((1, _D), lambda i: (0, 0)),
            pl.BlockSpec((1, 1, _BLK), lambda i: (i, 0, 0)),
            pl.BlockSpec((_D, _D), lambda i: (0, 0)),
            pl.BlockSpec((1, _D), lambda i: (0, 0)),
        ],
        out_specs=pl.BlockSpec((_G, _D), lambda i: (0, 0)),
        out_shape=jax.ShapeDtypeStruct((_G, _D), jnp.float32),
        scratch_shapes=[
            pltpu.VMEM((_G, _D), jnp.float32),
            pltpu.VMEM((_G, 1), jnp.float32),
        ],
    )(aggp, y2, dinv, b2, batch3d, wfc_pad, bfc_pad)


@jax.jit
def _run(x, src, dst, batch, w1, b1, w2, b2, w_fc, b_fc):
    npad_e = _EPAD - _E
    pad_src = (jnp.arange(npad_e, dtype=jnp.int32) * 13) % _N
    pad_dst = _N + (jnp.arange(npad_e, dtype=jnp.int32) % (_NPAD - _N))
    srcp = jnp.concatenate([src, pad_src]).reshape(_NW, _NCH, 1, _CHUNK)
    dstp = jnp.concatenate([dst, pad_dst]).reshape(_NW, _NCH, 1, _CHUNK)
    sd = jnp.concatenate([srcp, dstp], axis=2)
    dst3 = dstp.reshape(_NW, _NCH, _CHUNK)
    zeros1 = jnp.zeros((_NPAD,), jnp.float32)
    zeros2 = jnp.zeros((_NPAD, _D), jnp.float32)

    degp = _deg_kernel(dst3, zeros1)
    degp_col = degp.reshape(_NC, _NPAD, 1)[:, :_N]

    xw1 = _tc1a(x, w1)
    y1, dinv = _tc1b(xw1, degp_col)
    agg1 = _agg_kernel(y1, sd, zeros2)
    y2 = _tc2(agg1, y1, dinv, b1.reshape(1, _D), w2)
    agg2 = _agg_kernel(y2, sd, zeros2)

    wfc_pad = jnp.zeros((_D, _D), jnp.float32).at[:, :10].set(w_fc)
    bfc_pad = jnp.zeros((1, _D), jnp.float32).at[0, :10].set(b_fc)
    batch3d = batch.astype(jnp.int32).reshape(_NBLK, 1, _BLK)
    out = _tc3(agg2, y2, dinv, b2.reshape(1, _D), batch3d, wfc_pad, bfc_pad)
    return out[:, :10]


def kernel(x, edge_index, batch, W1, b1, W2, b2, W_fc, b_fc):
    ei = edge_index.astype(jnp.int32)
    return _run(x, ei[0], ei[1], batch, W1, b1, W2, b2, W_fc, b_fc)
